# baseline (speedup 1.0000x reference)
import jax
import jax.numpy as jnp
from jax import lax
from jax.experimental import pallas as pl
from jax.experimental.pallas import tpu as pltpu

_B = 1024
_DMLP = 2048
_DIM = 128
_K = 16384
_NS = 64


def _fold_sum_lanes(x):
    w = x.shape[-1]
    while w > 1:
        w //= 2
        x = x[:, :w] + x[:, w:]
    return x


def _ans_body(sidx_ref, imq_ref, w1_ref, b1_ref, w2_ref, b2_ref, qt_ref,
              invd_ref, ans_ref, s_ref):
    h = jnp.maximum(
        jnp.dot(imq_ref[...], w1_ref[...], preferred_element_type=jnp.float32)
        + b1_ref[...], 0.0)
    q = jnp.dot(h, w2_ref[...], preferred_element_type=jnp.float32) + b2_ref[...]
    nrm = jnp.sqrt(_fold_sum_lanes(q * q))
    q = q / jnp.maximum(nrm, 1e-12)

    def gbody(j, carry):
        idx = sidx_ref[j]
        s_ref[pl.ds(j, 1), :] = qt_ref[pl.ds(idx, 1), :]
        return carry

    lax.fori_loop(0, _NS, gbody, 0)

    invd = invd_ref[...]

    def dbody(n, acc):
        s_n = s_ref[pl.ds(n, 1), :]
        diff = q - s_n
        tmp = jnp.dot(diff, invd, preferred_element_type=jnp.float32)
        m = _fold_sum_lanes(tmp * diff)
        return acc + jnp.sqrt(jnp.maximum(m, 0.0))

    acc = lax.fori_loop(0, _NS, dbody, jnp.zeros((_B, 1), jnp.float32))
    ans_ref[...] = acc * (1.0 / _NS)


def _select_body(ansc_ref, ansr_ref, outp_ref, out_ref):
    a_row = ansr_ref[...]
    chunk = 32

    def cbody(i, carry):
        base = i * chunk
        a_col = ansc_ref[pl.ds(base, chunk), :]
        ii = base + lax.broadcasted_iota(jnp.int32, (chunk, _B), 0)
        jj = lax.broadcasted_iota(jnp.int32, (chunk, _B), 1)
        lt = a_row < a_col
        eqp = (a_row == a_col) & (jj < ii)
        cnt = _fold_sum_lanes(jnp.where(lt | eqp, 1.0, 0.0))
        sel = (cnt >= float(_B - 64)) & (cnt <= float(_B - 2))
        o = outp_ref[pl.ds(base, chunk), :]
        c = (jnp.abs(o[:, 2:3]) < 1.0) | (jnp.abs(o[:, 3:4]) < 1.0)
        msel = sel & c
        col = lax.broadcasted_iota(jnp.int32, (chunk, 8), 1)
        newo = jnp.where(msel & (col == 2), -5.0,
                         jnp.where(msel & (col == 3), 5.0, o))
        out_ref[pl.ds(base, chunk), :] = newo
        return carry

    lax.fori_loop(0, _B // chunk, cbody, 0)


def _compute_ans(im_q, W1, b1, W2, b2, queueT, invD, sample_idx):
    return pl.pallas_call(
        _ans_body,
        out_shape=jax.ShapeDtypeStruct((_B, 1), jnp.float32),
        in_specs=[
            pl.BlockSpec(memory_space=pltpu.SMEM),
            pl.BlockSpec(memory_space=pltpu.VMEM),
            pl.BlockSpec(memory_space=pltpu.VMEM),
            pl.BlockSpec(memory_space=pltpu.VMEM),
            pl.BlockSpec(memory_space=pltpu.VMEM),
            pl.BlockSpec(memory_space=pltpu.VMEM),
            pl.BlockSpec(memory_space=pltpu.VMEM),
            pl.BlockSpec(memory_space=pltpu.VMEM),
        ],
        scratch_shapes=[pltpu.VMEM((_NS, _DIM), jnp.float32)],
        compiler_params=pltpu.CompilerParams(
            vmem_limit_bytes=100 * 1024 * 1024),
    )(sample_idx, im_q, W1, b1, W2, b2, queueT, invD)


def _select_update(ans_col, ans_row, output):
    return pl.pallas_call(
        _select_body,
        out_shape=jax.ShapeDtypeStruct((_B, 8), jnp.float32),
    )(ans_col, ans_row, output)


def kernel(im_q, output, target, W1, b1, W2, b2, queue, invD, sample_idx):
    queueT = queue.T
    ans = _compute_ans(im_q, W1, b1, W2, b2, queueT, invD, sample_idx)
    return _select_update(ans, ans.reshape(1, _B), output)


# speedup vs baseline: 1.5987x; 1.5987x over previous
import functools

import jax
import jax.numpy as jnp
from jax import lax
from jax.experimental import pallas as pl
from jax.experimental.pallas import tpu as pltpu
from jax.experimental.pallas import tpu_sc as plsc

_B = 1024
_DMLP = 2048
_DIM = 128
_K = 16384
_NS = 64
_NBLK = 8
_BLK = _DMLP // _NBLK


def _fold_lanes(x):
    w = x.shape[-1]
    while w > 1:
        w //= 2
        x = x[:, :w] + x[:, w:]
    return x


def _fold_rows(x):
    h = x.shape[0]
    while h > 1:
        h //= 2
        x = x[:h] + x[h:]
    return x


@functools.lru_cache(maxsize=1)
def _make_sc_gather():
    mesh = plsc.VectorSubcoreMesh(core_axis_name="c", subcore_axis_name="s")

    @functools.partial(
        pl.kernel,
        out_type=jax.ShapeDtypeStruct((_NS, _DIM), jnp.float32),
        mesh=mesh,
        scratch_types=[
            pltpu.VMEM((_NS + 16,), jnp.int32),
            pltpu.VMEM((_DIM,), jnp.int32),
            pltpu.VMEM((_DIM,), jnp.int32),
            pltpu.VMEM((_DIM,), jnp.float32),
            pltpu.VMEM((_DIM,), jnp.float32),
            pltpu.SemaphoreType.DMA,
        ],
    )
    def sc_gather(qflat_hbm, sidx_hbm, out_hbm, sidx_v, ixa, ixb, rowa, rowb,
                  sem):
        wid = lax.axis_index("s") * 2 + lax.axis_index("c")
        pltpu.sync_copy(sidx_hbm, sidx_v.at[pl.ds(0, _NS)])
        for j2 in range(2):
            j = wid * 2 + j2
            sj = sidx_v[pl.ds(j, 16)][0]
            ix = (ixa, ixb)[j2]
            for v in range(8):
                lane = lax.iota(jnp.int32, 16)
                ix[pl.ds(16 * v, 16)] = (lane + 16 * v) * _K + sj
            buf = (rowa, rowb)[j2]
            pltpu.async_copy(qflat_hbm.at[ix], buf, sem).wait()
            pltpu.sync_copy(buf, out_hbm.at[j])

    return sc_gather


def _sc_gather(qflat, sidx):
    return _make_sc_gather()(qflat, sidx)


def _mlp_body(imq_ref, w1_ref, b1_ref, w2_ref, b2_ref, q_ref):
    j = pl.program_id(0)
    h = jnp.maximum(
        jnp.dot(imq_ref[...], w1_ref[...], preferred_element_type=jnp.float32)
        + b1_ref[0], 0.0)
    part = jnp.dot(h, w2_ref[...], preferred_element_type=jnp.float32)

    @pl.when(j == 0)
    def _():
        q_ref[...] = part

    @pl.when(j > 0)
    def _():
        q_ref[...] += part

    @pl.when(j == _NBLK - 1)
    def _():
        q = q_ref[...] + b2_ref[...]
        nrm = jnp.sqrt(_fold_lanes(q * q))
        q_ref[...] = q / jnp.maximum(nrm, 1e-12)


def _mlp(im_q, W1, b1r, W2, b2r):
    return pl.pallas_call(
        _mlp_body,
        grid=(_NBLK,),
        in_specs=[
            pl.BlockSpec((_B, _DMLP), lambda j: (0, 0)),
            pl.BlockSpec((_DMLP, _BLK), lambda j: (0, j)),
            pl.BlockSpec((1, 1, _BLK), lambda j: (j, 0, 0)),
            pl.BlockSpec((_BLK, _DIM), lambda j: (j, 0)),
            pl.BlockSpec((1, _DIM), lambda j: (0, 0)),
        ],
        out_specs=pl.BlockSpec((_B, _DIM), lambda j: (0, 0)),
        out_shape=jax.ShapeDtypeStruct((_B, _DIM), jnp.float32),
        compiler_params=pltpu.CompilerParams(
            dimension_semantics=("arbitrary",),
            vmem_limit_bytes=100 * 1024 * 1024),
    )(im_q, W1, b1r, W2, b2r)


def _dist_body(qt_ref, srows_ref, invdt_ref, ans_ref, m_ref):
    qT = qt_ref[...]
    A = invdt_ref[...]

    def dbody(n, c):
        col = srows_ref[pl.ds(n, 1), :].reshape(_DIM, 1)
        diffT = qT - col
        tmpT = jnp.dot(A, diffT, preferred_element_type=jnp.float32)
        m = jnp.sum(tmpT * diffT, axis=0, keepdims=True)
        m_ref[pl.ds(n, 1), :] = jnp.sqrt(jnp.maximum(m, 0.0))
        return c

    lax.fori_loop(0, _NS, dbody, 0)
    ans_ref[...] = _fold_rows(m_ref[...]) * (1.0 / _NS)


def _dist(qT, srows, invDT):
    return pl.pallas_call(
        _dist_body,
        out_shape=jax.ShapeDtypeStruct((1, _B), jnp.float32),
        scratch_shapes=[pltpu.VMEM((_NS, _B), jnp.float32)],
        compiler_params=pltpu.CompilerParams(
            vmem_limit_bytes=32 * 1024 * 1024),
    )(qT, srows, invDT)


def _select_body(ansc_ref, ansr_ref, outp_ref, out_ref):
    a_row = ansr_ref[...]
    chunk = 32

    def cbody(i, carry):
        base = i * chunk
        a_col = ansc_ref[pl.ds(base, chunk), :]
        ii = base + lax.broadcasted_iota(jnp.int32, (chunk, _B), 0)
        jj = lax.broadcasted_iota(jnp.int32, (chunk, _B), 1)
        lt = a_row < a_col
        eqp = (a_row == a_col) & (jj < ii)
        cnt = _fold_lanes(jnp.where(lt | eqp, 1.0, 0.0))
        sel = (cnt >= float(_B - 64)) & (cnt <= float(_B - 2))
        o = outp_ref[pl.ds(base, chunk), :]
        c = (jnp.abs(o[:, 2:3]) < 1.0) | (jnp.abs(o[:, 3:4]) < 1.0)
        msel = sel & c
        col = lax.broadcasted_iota(jnp.int32, (chunk, 8), 1)
        newo = jnp.where(msel & (col == 2), -5.0,
                         jnp.where(msel & (col == 3), 5.0, o))
        out_ref[pl.ds(base, chunk), :] = newo
        return carry

    lax.fori_loop(0, _B // chunk, cbody, 0)


def _select_update(ans_col, ans_row, output):
    return pl.pallas_call(
        _select_body,
        out_shape=jax.ShapeDtypeStruct((_B, 8), jnp.float32),
    )(ans_col, ans_row, output)


def kernel(im_q, output, target, W1, b1, W2, b2, queue, invD, sample_idx):
    srows = _sc_gather(queue.reshape(-1), sample_idx)
    q = _mlp(im_q, W1, b1.reshape(_NBLK, 1, _BLK), W2, b2.reshape(1, _DIM))
    ans_row = _dist(q.T, srows, invD.T)
    return _select_update(ans_row.reshape(_B, 1), ans_row, output)


# speedup vs baseline: 2.0587x; 1.2877x over previous
import functools

import jax
import jax.numpy as jnp
from jax import lax
from jax.experimental import pallas as pl
from jax.experimental.pallas import tpu as pltpu
from jax.experimental.pallas import tpu_sc as plsc

_B = 1024
_DMLP = 2048
_DIM = 128
_K = 16384
_NS = 64
_NBLK = 8
_BLK = _DMLP // _NBLK


def _fold_lanes(x):
    w = x.shape[-1]
    while w > 1:
        w //= 2
        x = x[:, :w] + x[:, w:]
    return x


def _fold_rows(x):
    h = x.shape[0]
    while h > 1:
        h //= 2
        x = x[:h] + x[h:]
    return x


@functools.lru_cache(maxsize=1)
def _make_sc_gather():
    mesh = plsc.VectorSubcoreMesh(core_axis_name="c", subcore_axis_name="s")
    rows_per = _DIM // 32

    @functools.partial(
        pl.kernel,
        out_type=jax.ShapeDtypeStruct((_DIM, _NS), jnp.float32),
        mesh=mesh,
        scratch_types=[
            pltpu.VMEM((_NS,), jnp.int32),
            [pltpu.VMEM((_K,), jnp.float32)] * rows_per,
            [pltpu.VMEM((_NS,), jnp.float32)] * rows_per,
            pltpu.SemaphoreType.DMA,
        ],
        compiler_params=pltpu.CompilerParams(needs_layout_passes=False),
    )
    def sc_gather(queue_hbm, sidx_hbm, out_hbm, sidx_v, rows_v, res_v, sem):
        wid = lax.axis_index("s") * 2 + lax.axis_index("c")
        base = wid * rows_per
        cps = [pltpu.async_copy(queue_hbm.at[base + r], rows_v[r], sem)
               for r in range(rows_per)]
        pltpu.sync_copy(sidx_hbm, sidx_v)
        for cp in cps:
            cp.wait()
        for r in range(rows_per):
            for c in range(_NS // 16):
                idx = sidx_v[pl.ds(16 * c, 16)]
                vals = plsc.load_gather(rows_v[r], [idx])
                res_v[r][pl.ds(16 * c, 16)] = vals
            pltpu.sync_copy(res_v[r], out_hbm.at[base + r])

    return sc_gather


def _sc_gather(queue, sidx):
    return _make_sc_gather()(queue, sidx)


def _mlp_body(imq_ref, w1_ref, b1_ref, w2_ref, b2_ref, q_ref):
    j = pl.program_id(0)
    h = jnp.maximum(
        jnp.dot(imq_ref[...], w1_ref[...], preferred_element_type=jnp.float32)
        + b1_ref[0], 0.0)
    part = jnp.dot(h, w2_ref[...], preferred_element_type=jnp.float32)

    @pl.when(j == 0)
    def _():
        q_ref[...] = part

    @pl.when(j > 0)
    def _():
        q_ref[...] += part

    @pl.when(j == _NBLK - 1)
    def _():
        q = q_ref[...] + b2_ref[...]
        nrm = jnp.sqrt(_fold_lanes(q * q))
        q_ref[...] = q / jnp.maximum(nrm, 1e-12)


def _mlp(im_q, W1, b1r, W2, b2r):
    return pl.pallas_call(
        _mlp_body,
        grid=(_NBLK,),
        in_specs=[
            pl.BlockSpec((_B, _DMLP), lambda j: (0, 0)),
            pl.BlockSpec((_DMLP, _BLK), lambda j: (0, j)),
            pl.BlockSpec((1, 1, _BLK), lambda j: (j, 0, 0)),
            pl.BlockSpec((_BLK, _DIM), lambda j: (j, 0)),
            pl.BlockSpec((1, _DIM), lambda j: (0, 0)),
        ],
        out_specs=pl.BlockSpec((_B, _DIM), lambda j: (0, 0)),
        out_shape=jax.ShapeDtypeStruct((_B, _DIM), jnp.float32),
        compiler_params=pltpu.CompilerParams(
            dimension_semantics=("arbitrary",),
            vmem_limit_bytes=100 * 1024 * 1024),
    )(im_q, W1, b1r, W2, b2r)


def _dist_body(qt_ref, srows_ref, invdt_ref, ans_ref, m_ref):
    qT = qt_ref[...]
    A = invdt_ref[...]

    def dbody(n, c):
        col = srows_ref[pl.ds(n, 1), :].reshape(_DIM, 1)
        diffT = qT - col
        tmpT = jnp.dot(A, diffT, preferred_element_type=jnp.float32)
        m = jnp.sum(tmpT * diffT, axis=0, keepdims=True)
        m_ref[pl.ds(n, 1), :] = jnp.sqrt(jnp.maximum(m, 0.0))
        return c

    lax.fori_loop(0, _NS, dbody, 0)
    ans_ref[...] = _fold_rows(m_ref[...]) * (1.0 / _NS)


def _dist(qT, srows, invDT):
    return pl.pallas_call(
        _dist_body,
        out_shape=jax.ShapeDtypeStruct((1, _B), jnp.float32),
        scratch_shapes=[pltpu.VMEM((_NS, _B), jnp.float32)],
        compiler_params=pltpu.CompilerParams(
            vmem_limit_bytes=32 * 1024 * 1024),
    )(qT, srows, invDT)


def _select_body(ansc_ref, ansr_ref, outp_ref, out_ref):
    a_row = ansr_ref[...]
    chunk = 128

    def cbody(i, carry):
        base = i * chunk
        a_col = ansc_ref[pl.ds(base, chunk), :]
        ii = base + lax.broadcasted_iota(jnp.int32, (chunk, _B), 0)
        jj = lax.broadcasted_iota(jnp.int32, (chunk, _B), 1)
        lt = a_row < a_col
        eqp = (a_row == a_col) & (jj < ii)
        cnt = _fold_lanes(jnp.where(lt | eqp, 1.0, 0.0))
        sel = (cnt >= float(_B - 64)) & (cnt <= float(_B - 2))
        o = outp_ref[pl.ds(base, chunk), :]
        c = (jnp.abs(o[:, 2:3]) < 1.0) | (jnp.abs(o[:, 3:4]) < 1.0)
        msel = sel & c
        col = lax.broadcasted_iota(jnp.int32, (chunk, 8), 1)
        newo = jnp.where(msel & (col == 2), -5.0,
                         jnp.where(msel & (col == 3), 5.0, o))
        out_ref[pl.ds(base, chunk), :] = newo
        return carry

    lax.fori_loop(0, _B // chunk, cbody, 0)


def _select_update(ans_col, ans_row, output):
    return pl.pallas_call(
        _select_body,
        out_shape=jax.ShapeDtypeStruct((_B, 8), jnp.float32),
    )(ans_col, ans_row, output)


def kernel(im_q, output, target, W1, b1, W2, b2, queue, invD, sample_idx):
    srows = _sc_gather(queue, sample_idx).T
    q = _mlp(im_q, W1, b1.reshape(_NBLK, 1, _BLK), W2, b2.reshape(1, _DIM))
    ans_row = _dist(q.T, srows, invD.T)
    return _select_update(ans_row.reshape(_B, 1), ans_row, output)


# speedup vs baseline: 2.7236x; 1.3230x over previous
import functools

import jax
import jax.numpy as jnp
from jax import lax
from jax.experimental import pallas as pl
from jax.experimental.pallas import tpu as pltpu
from jax.experimental.pallas import tpu_sc as plsc

_B = 1024
_DMLP = 2048
_DIM = 128
_K = 16384
_NS = 64
_NBLK = 8
_BLK = _DMLP // _NBLK


def _fold_lanes(x):
    w = x.shape[-1]
    while w > 1:
        w //= 2
        x = x[:, :w] + x[:, w:]
    return x


def _fold_rows(x):
    h = x.shape[0]
    while h > 1:
        h //= 2
        x = x[:h] + x[h:]
    return x


@functools.lru_cache(maxsize=1)
def _make_sc_gather():
    mesh = plsc.VectorSubcoreMesh(core_axis_name="c", subcore_axis_name="s")
    rows_per = _DIM // 32

    @functools.partial(
        pl.kernel,
        out_type=jax.ShapeDtypeStruct((_DIM, _NS), jnp.float32),
        mesh=mesh,
        scratch_types=[
            pltpu.VMEM((_NS,), jnp.int32),
            [pltpu.VMEM((_K,), jnp.float32)] * rows_per,
            [pltpu.VMEM((_NS,), jnp.float32)] * rows_per,
            pltpu.SemaphoreType.DMA,
        ],
        compiler_params=pltpu.CompilerParams(needs_layout_passes=False),
    )
    def sc_gather(queue_hbm, sidx_hbm, out_hbm, sidx_v, rows_v, res_v, sem):
        wid = lax.axis_index("s") * 2 + lax.axis_index("c")
        base = wid * rows_per
        cps = [pltpu.async_copy(queue_hbm.at[base + r], rows_v[r], sem)
               for r in range(rows_per)]
        pltpu.sync_copy(sidx_hbm, sidx_v)
        for cp in cps:
            cp.wait()
        for r in range(rows_per):
            for c in range(_NS // 16):
                idx = sidx_v[pl.ds(16 * c, 16)]
                vals = plsc.load_gather(rows_v[r], [idx])
                res_v[r][pl.ds(16 * c, 16)] = vals
            pltpu.sync_copy(res_v[r], out_hbm.at[base + r])

    return sc_gather


def _sc_gather(queue, sidx):
    return _make_sc_gather()(queue, sidx)


def _mlp_body(imq_ref, w1_ref, b1_ref, w2_ref, b2_ref, q_ref):
    j = pl.program_id(0)
    h = jnp.maximum(
        jnp.dot(imq_ref[...], w1_ref[...], preferred_element_type=jnp.float32)
        + b1_ref[0], 0.0)
    part = jnp.dot(h, w2_ref[...], preferred_element_type=jnp.float32)

    @pl.when(j == 0)
    def _():
        q_ref[...] = part

    @pl.when(j > 0)
    def _():
        q_ref[...] += part

    @pl.when(j == _NBLK - 1)
    def _():
        q = q_ref[...] + b2_ref[...]
        nrm = jnp.sqrt(_fold_lanes(q * q))
        q_ref[...] = q / jnp.maximum(nrm, 1e-12)


def _mlp(im_q, W1, b1r, W2, b2r):
    return pl.pallas_call(
        _mlp_body,
        grid=(_NBLK,),
        in_specs=[
            pl.BlockSpec((_B, _DMLP), lambda j: (0, 0)),
            pl.BlockSpec((_DMLP, _BLK), lambda j: (0, j)),
            pl.BlockSpec((1, 1, _BLK), lambda j: (j, 0, 0)),
            pl.BlockSpec((_BLK, _DIM), lambda j: (j, 0)),
            pl.BlockSpec((1, _DIM), lambda j: (0, 0)),
        ],
        out_specs=pl.BlockSpec((_B, _DIM), lambda j: (0, 0)),
        out_shape=jax.ShapeDtypeStruct((_B, _DIM), jnp.float32),
        compiler_params=pltpu.CompilerParams(
            dimension_semantics=("arbitrary",),
            vmem_limit_bytes=100 * 1024 * 1024),
    )(im_q, W1, b1r, W2, b2r)


def _dist_sel_body(qt_ref, srows_ref, invdt_ref, outp_ref, out_ref,
                   m_ref, acol_ref):
    qT = qt_ref[...]
    A = invdt_ref[...]

    def dbody(i, c):
        s4 = srows_ref[pl.ds(4 * i, 4), :]
        diff4 = jnp.concatenate(
            [qT - s4[j:j + 1, :].reshape(_DIM, 1) for j in range(4)], axis=1)
        tmp4 = jnp.dot(A, diff4, preferred_element_type=jnp.float32)
        m4 = jnp.sum(tmp4 * diff4, axis=0, keepdims=True)
        sq = jnp.sqrt(jnp.maximum(m4, 0.0))
        for j in range(4):
            m_ref[pl.ds(4 * i + j, 1), :] = sq[:, _B * j:_B * (j + 1)]
        return c

    lax.fori_loop(0, _NS // 4, dbody, 0)
    a_row = _fold_rows(m_ref[...]) * (1.0 / _NS)
    acol_ref[...] = a_row.reshape(_B, 1)
    ones = jnp.ones((_B, 8), jnp.float32)
    chunk = 128

    def cbody(i, carry):
        base = i * chunk
        a_col = acol_ref[pl.ds(base, chunk), :]
        ii = base + lax.broadcasted_iota(jnp.int32, (chunk, _B), 0)
        jj = lax.broadcasted_iota(jnp.int32, (chunk, _B), 1)
        lt = a_row < a_col
        eqp = (a_row == a_col) & (jj < ii)
        ind = jnp.where(lt | eqp, 1.0, 0.0)
        cnt = jnp.dot(ind, ones, preferred_element_type=jnp.float32)[:, 0:1]
        sel = (cnt >= float(_B - 64)) & (cnt <= float(_B - 2))
        o = outp_ref[pl.ds(base, chunk), :]
        c = (jnp.abs(o[:, 2:3]) < 1.0) | (jnp.abs(o[:, 3:4]) < 1.0)
        msel = sel & c
        col = lax.broadcasted_iota(jnp.int32, (chunk, 8), 1)
        newo = jnp.where(msel & (col == 2), -5.0,
                         jnp.where(msel & (col == 3), 5.0, o))
        out_ref[pl.ds(base, chunk), :] = newo
        return carry

    lax.fori_loop(0, _B // chunk, cbody, 0)


def _dist_select(qT, srows, invDT, output):
    return pl.pallas_call(
        _dist_sel_body,
        out_shape=jax.ShapeDtypeStruct((_B, 8), jnp.float32),
        scratch_shapes=[pltpu.VMEM((_NS, _B), jnp.float32),
                        pltpu.VMEM((_B, 1), jnp.float32)],
        compiler_params=pltpu.CompilerParams(
            vmem_limit_bytes=32 * 1024 * 1024),
    )(qT, srows, invDT, output)


def kernel(im_q, output, target, W1, b1, W2, b2, queue, invD, sample_idx):
    srows = _sc_gather(queue, sample_idx).T
    q = _mlp(im_q, W1, b1.reshape(_NBLK, 1, _BLK), W2, b2.reshape(1, _DIM))
    return _dist_select(q.T, srows, invD.T, output)
